# baseline (speedup 1.0000x reference)
import functools

import jax
import jax.numpy as jnp
from jax.experimental import pallas as pl
from jax.experimental.pallas import tpu as pltpu

B, S, D, H, E, K, M = 1, 2048, 768, 1536, 8, 2, 4
ROWT = 256
CAP = S
TPG = CAP // ROWT
NSTEP_R = E * K * S // ROWT + E
NSTEP_M = S // ROWT + M
SQ2I = 0.7071067811865476


def _gelu(v):
    return 0.5 * v * (1.0 + jax.lax.erf(v * SQ2I))


def _router_kernel(x_ref, mode_ref, rw_ref, rb_ref, logits_ref, tidx_ref,
                   tprob_ref, sgr_ref, sbr_ref, totr_ref, sgm_ref, sbm_ref,
                   totm_ref):
    x = x_ref[...]
    logits = jax.lax.dot_general(x, rw_ref[...], (((1,), (1,)), ((), ())),
                                 preferred_element_type=jnp.float32)
    logits = logits + rb_ref[...][None, :]
    logits_ref[...] = logits
    mx = jnp.max(logits, axis=1, keepdims=True)
    ex = jnp.exp(logits - mx)
    probs = ex / jnp.sum(ex, axis=1, keepdims=True)
    iota = jax.lax.broadcasted_iota(jnp.int32, (S, E), 1)
    p0 = jnp.max(probs, axis=1, keepdims=True)
    i0 = jnp.min(jnp.where(probs == p0, iota, E), axis=1, keepdims=True)
    masked = jnp.where(iota == i0, -jnp.inf, probs)
    p1 = jnp.max(masked, axis=1, keepdims=True)
    i1 = jnp.min(jnp.where(masked == p1, iota, E), axis=1, keepdims=True)
    tidx_ref[...] = jnp.concatenate([i0, i1], axis=1)
    tprob_ref[...] = jnp.concatenate([p0, p1], axis=1)

    def steps(counts_col, n_g, n_step, sg_ref, sb_ref, tot_ref):
        ti = jax.lax.broadcasted_iota(jnp.int32, (n_g, n_g), 0)
        tj = jax.lax.broadcasted_iota(jnp.int32, (n_g, n_g), 1)
        tri = (tj <= ti).astype(jnp.float32)
        nt_col = jnp.floor((counts_col + float(ROWT - 1)) / float(ROWT))
        cum_col = jax.lax.dot_general(tri, nt_col, (((1,), (0,)), ((), ())),
                                      preferred_element_type=jnp.float32)
        cumprev_col = cum_col - nt_col
        total = jnp.sum(nt_col, axis=0, keepdims=True)
        s = jax.lax.broadcasted_iota(jnp.int32, (1, n_step),
                                     1).astype(jnp.float32)
        s_eff = jnp.minimum(s, jnp.maximum(total - 1.0, 0.0))
        ge = (s_eff >= cum_col).astype(jnp.float32)
        g_of_s = jnp.sum(ge, axis=0, keepdims=True)
        gi = jax.lax.broadcasted_iota(jnp.int32, (n_g, n_step),
                                      0).astype(jnp.float32)
        onehot = (gi == g_of_s).astype(jnp.float32)
        cumprev_of_s = jnp.sum(cumprev_col * onehot, axis=0, keepdims=True)
        j_of_s = s_eff - cumprev_of_s
        sg_ref[...] = g_of_s.astype(jnp.int32)
        sb_ref[...] = (g_of_s * TPG + j_of_s).astype(jnp.int32)
        tot_ref[...] = total.astype(jnp.int32)

    ones_col = jnp.ones((S, 1), jnp.float32)
    oh_r = ((iota == i0) | (iota == i1)).astype(jnp.float32)
    counts_r = jax.lax.dot_general(oh_r, ones_col, (((0,), (0,)), ((), ())),
                                   preferred_element_type=jnp.float32)
    steps(counts_r, E, NSTEP_R, sgr_ref, sbr_ref, totr_ref)

    miota = jax.lax.broadcasted_iota(jnp.int32, (S, M), 1)
    oh_m = (mode_ref[...] == miota).astype(jnp.float32)
    counts_m = jax.lax.dot_general(oh_m, ones_col, (((0,), (0,)), ((), ())),
                                   preferred_element_type=jnp.float32)
    steps(counts_m, M, NSTEP_M, sgm_ref, sbm_ref, totm_ref)


def _run_router(flat, mode_flat, router_w, router_b):
    return pl.pallas_call(
        _router_kernel,
        grid=(1,),
        in_specs=[
            pl.BlockSpec((S, D), lambda i: (0, 0)),
            pl.BlockSpec((S, 1), lambda i: (0, 0)),
            pl.BlockSpec((E, D), lambda i: (0, 0)),
            pl.BlockSpec((E,), lambda i: (0,)),
        ],
        out_specs=[
            pl.BlockSpec((S, E), lambda i: (0, 0)),
            pl.BlockSpec((S, K), lambda i: (0, 0)),
            pl.BlockSpec((S, K), lambda i: (0, 0)),
            pl.BlockSpec((1, NSTEP_R), lambda i: (0, 0)),
            pl.BlockSpec((1, NSTEP_R), lambda i: (0, 0)),
            pl.BlockSpec((1, 1), lambda i: (0, 0)),
            pl.BlockSpec((1, NSTEP_M), lambda i: (0, 0)),
            pl.BlockSpec((1, NSTEP_M), lambda i: (0, 0)),
            pl.BlockSpec((1, 1), lambda i: (0, 0)),
        ],
        out_shape=[
            jax.ShapeDtypeStruct((S, E), jnp.float32),
            jax.ShapeDtypeStruct((S, K), jnp.int32),
            jax.ShapeDtypeStruct((S, K), jnp.float32),
            jax.ShapeDtypeStruct((1, NSTEP_R), jnp.int32),
            jax.ShapeDtypeStruct((1, NSTEP_R), jnp.int32),
            jax.ShapeDtypeStruct((1, 1), jnp.int32),
            jax.ShapeDtypeStruct((1, NSTEP_M), jnp.int32),
            jax.ShapeDtypeStruct((1, NSTEP_M), jnp.int32),
            jax.ShapeDtypeStruct((1, 1), jnp.int32),
        ],
    )(flat, mode_flat, router_w, router_b)


def _shared_ffn_kernel(x_ref, w1_ref, b1_ref, w2_ref, b2_ref, out_ref):
    h = jax.lax.dot_general(x_ref[...], w1_ref[0], (((1,), (1,)), ((), ())),
                            preferred_element_type=jnp.float32)
    h = _gelu(h + b1_ref[0])
    y = jax.lax.dot_general(h, w2_ref[0], (((1,), (1,)), ((), ())),
                            preferred_element_type=jnp.float32)
    out_ref[...] = y + b2_ref[0]


def _run_shared(flat, w1, b1, w2, b2):
    tt = 1024
    return pl.pallas_call(
        _shared_ffn_kernel,
        grid=(S // tt,),
        in_specs=[
            pl.BlockSpec((tt, D), lambda i: (i, 0)),
            pl.BlockSpec((1, H, D), lambda i: (0, 0, 0)),
            pl.BlockSpec((1, 1, H), lambda i: (0, 0, 0)),
            pl.BlockSpec((1, D, H), lambda i: (0, 0, 0)),
            pl.BlockSpec((1, 1, D), lambda i: (0, 0, 0)),
        ],
        out_specs=pl.BlockSpec((tt, D), lambda i: (i, 0)),
        out_shape=jax.ShapeDtypeStruct((S, D), jnp.float32),
    )(flat, w1, b1.reshape(1, 1, H), w2, b2.reshape(1, 1, D))


def _grouped_ffn_kernel(sg_ref, sb_ref, tot_ref, x_ref, w1_ref, b1_ref,
                        w2_ref, b2_ref, wrow_ref, y_ref):
    s = pl.program_id(0)

    @pl.when(s < tot_ref[0])
    def _():
        h = jax.lax.dot_general(x_ref[...], w1_ref[0],
                                (((1,), (1,)), ((), ())),
                                preferred_element_type=jnp.float32)
        h = _gelu(h + b1_ref[0])
        y = jax.lax.dot_general(h, w2_ref[0], (((1,), (1,)), ((), ())),
                                preferred_element_type=jnp.float32)
        y_ref[...] = (y + b2_ref[0]) * wrow_ref[0]


def _run_grouped(xg, wrow, w1, b1, w2, b2, sg, sb, tot, n_g, n_step):
    nrow = n_g * CAP
    grid_spec = pltpu.PrefetchScalarGridSpec(
        num_scalar_prefetch=3,
        grid=(n_step,),
        in_specs=[
            pl.BlockSpec((ROWT, D), lambda s, sg, sb, tot: (sb[s], 0)),
            pl.BlockSpec((1, H, D), lambda s, sg, sb, tot: (sg[s], 0, 0)),
            pl.BlockSpec((1, 1, H), lambda s, sg, sb, tot: (sg[s], 0, 0)),
            pl.BlockSpec((1, D, H), lambda s, sg, sb, tot: (sg[s], 0, 0)),
            pl.BlockSpec((1, 1, D), lambda s, sg, sb, tot: (sg[s], 0, 0)),
            pl.BlockSpec((1, ROWT, 1), lambda s, sg, sb, tot: (sb[s], 0, 0)),
        ],
        out_specs=pl.BlockSpec((ROWT, D), lambda s, sg, sb, tot: (sb[s], 0)),
        scratch_shapes=[],
    )
    return pl.pallas_call(
        _grouped_ffn_kernel,
        grid_spec=grid_spec,
        out_shape=jax.ShapeDtypeStruct((nrow, D), jnp.float32),
    )(sg, sb, tot, xg, w1, b1.reshape(n_g, 1, H), w2,
      b2.reshape(n_g, 1, D), wrow.reshape(nrow // ROWT, ROWT, 1))


def _combine_kernel(a_ref, b_ref, c_ref, out_ref):
    out_ref[...] = a_ref[...] + b_ref[...] + c_ref[...]


def _run_combine(a, b, c):
    return pl.pallas_call(
        _combine_kernel,
        grid=(1,),
        in_specs=[pl.BlockSpec((S, D), lambda i: (0, 0))] * 3,
        out_specs=pl.BlockSpec((S, D), lambda i: (0, 0)),
        out_shape=jax.ShapeDtypeStruct((S, D), jnp.float32),
    )(a, b, c)


def _jnp_dispatch(flat, keys, weights, n_g):
    n_assign = keys.shape[0]
    per_tok = n_assign // S
    a = jnp.arange(n_assign, dtype=jnp.int32)
    order = jnp.argsort(keys * n_assign + a)
    sorted_tok = order // per_tok
    sorted_w = weights[order]
    sorted_key = keys[order]
    counts = jnp.sum(keys[None, :] == jnp.arange(n_g, dtype=jnp.int32)[:, None],
                     axis=1)
    offprev = jnp.cumsum(counts) - counts
    p = jnp.arange(n_assign, dtype=jnp.int32)
    dst = sorted_key * CAP + (p - offprev[sorted_key])
    nrow = n_g * CAP
    xg = jnp.zeros((nrow, D), jnp.float32).at[dst].set(flat[sorted_tok])
    wrow = jnp.zeros((nrow,), jnp.float32).at[dst].set(sorted_w)
    src = jnp.zeros((nrow,), jnp.int32).at[dst].set(sorted_tok)
    cpad = ((counts + ROWT - 1) // ROWT) * ROWT
    rowid = jnp.arange(nrow, dtype=jnp.int32)
    valid = (rowid % CAP) < cpad[rowid // CAP]
    return xg, wrow, src, valid


def kernel(hidden, mode_ids, router_w, router_b, shared_w1, shared_b1,
           shared_w2, shared_b2, routed_w1, routed_b1, routed_w2, routed_b2,
           mode_w1, mode_b1, mode_w2, mode_b2):
    flat = hidden.reshape(S, D)
    mode_flat = mode_ids.reshape(S, 1).astype(jnp.int32)

    (logits, tidx, tprob, sgr, sbr, totr, sgm, sbm,
     totm) = _run_router(flat, mode_flat, router_w, router_b)

    shared_out = _run_shared(flat, shared_w1, shared_b1, shared_w2, shared_b2)

    xg_r, wrow_r, src_r, valid_r = _jnp_dispatch(
        flat, tidx.reshape(-1), tprob.reshape(-1), E)
    xg_m, wrow_m, src_m, valid_m = _jnp_dispatch(
        flat, mode_flat.reshape(-1), jnp.ones((S,), jnp.float32), M)

    y_r = _run_grouped(xg_r, wrow_r, routed_w1, routed_b1, routed_w2,
                       routed_b2, sgr.reshape(-1), sbr.reshape(-1),
                       totr.reshape(-1), E, NSTEP_R)
    y_m = _run_grouped(xg_m, wrow_m, mode_w1, mode_b1, mode_w2, mode_b2,
                       sgm.reshape(-1), sbm.reshape(-1),
                       totm.reshape(-1), M, NSTEP_M)

    y_r = jnp.where(valid_r[:, None], y_r, 0.0)
    y_m = jnp.where(valid_m[:, None], y_m, 0.0)
    p0 = jnp.zeros((S, D), jnp.float32).at[src_r].add(y_r)
    p1 = jnp.zeros((S, D), jnp.float32).at[src_m].add(y_m)

    out = _run_combine(shared_out, p0, p1)
    return (out.reshape(B, S, D), logits.reshape(B, S, E),
            tidx.reshape(B, S, K), tprob.reshape(B, S, K))


# speedup vs baseline: 3.3399x; 3.3399x over previous
import functools

import jax
import jax.numpy as jnp
from jax import lax
from jax.experimental import pallas as pl
from jax.experimental.pallas import tpu as pltpu
from jax.experimental.pallas import tpu_sc as plsc

B, S, D, H, E, K, M = 1, 2048, 768, 1536, 8, 2, 4
ROWT = 256
CAP = S
TPG = CAP // ROWT
NSTEP_R = E * K * S // ROWT + E
NSTEP_M = S // ROWT + M
NROW_R = E * CAP
NROW_M = M * CAP
NA_R = K * S
NA_M = S
RB = 512
CHUNK = 128
CH2 = 64
WPAD = 128
SQ2I = 0.7071067811865476


def _gelu(v):
    return 0.5 * v * (1.0 + jax.lax.erf(v * SQ2I))


def _router_kernel(x_ref, mode_ref, rw_ref, rb_ref, logits_ref, tidx_ref,
                   tprob_ref, sgr_ref, sbr_ref, totr_ref, sgm_ref, sbm_ref,
                   totm_ref, dstr_ref, wr2d_ref, dstm_ref):
    x = x_ref[...]
    logits = jax.lax.dot_general(x, rw_ref[...], (((1,), (1,)), ((), ())),
                                 preferred_element_type=jnp.float32)
    logits = logits + rb_ref[...][None, :]
    logits_ref[...] = logits
    mx = jnp.max(logits, axis=1, keepdims=True)
    ex = jnp.exp(logits - mx)
    probs = ex / jnp.sum(ex, axis=1, keepdims=True)
    iota = jax.lax.broadcasted_iota(jnp.int32, (S, E), 1)
    p0 = jnp.max(probs, axis=1, keepdims=True)
    i0 = jnp.min(jnp.where(probs == p0, iota, E), axis=1, keepdims=True)
    masked = jnp.where(iota == i0, -jnp.inf, probs)
    p1 = jnp.max(masked, axis=1, keepdims=True)
    i1 = jnp.min(jnp.where(masked == p1, iota, E), axis=1, keepdims=True)
    tidx_ref[...] = jnp.concatenate([i0, i1], axis=1)
    tprob_ref[...] = jnp.concatenate([p0, p1], axis=1)

    def ranks(oh, n_a):
        ii = jax.lax.broadcasted_iota(jnp.int32, (RB, RB), 0)
        jj = jax.lax.broadcasted_iota(jnp.int32, (RB, RB), 1)
        tri = (jj < ii).astype(jnp.float32)
        n_g = oh.shape[1]
        running = jnp.zeros((1, n_g), jnp.float32)
        outs = []
        for b in range(n_a // RB):
            oh_b = oh[b * RB:(b + 1) * RB]
            local = jax.lax.dot_general(tri, oh_b, (((1,), (0,)), ((), ())),
                                        preferred_element_type=jnp.float32)
            outs.append(local + running)
            running = running + jnp.sum(oh_b, axis=0, keepdims=True)
        rank = jnp.concatenate(outs, axis=0)
        lanes = jax.lax.broadcasted_iota(jnp.int32, (n_a, n_g),
                                         1).astype(jnp.float32)
        rank_col = jnp.sum(rank * oh, axis=1, keepdims=True)
        e_col = jnp.sum(lanes * oh, axis=1, keepdims=True)
        dst = e_col * float(CAP) + rank_col
        return dst.astype(jnp.int32), running

    oh_r = jnp.concatenate([(iota == i0).astype(jnp.float32),
                            (iota == i1).astype(jnp.float32)], axis=0)
    dst_r, counts_r = ranks(oh_r, NA_R)
    dstr_ref[...] = dst_r
    w_col = jnp.concatenate([p0, p1], axis=0)
    wr2d_ref[...] = jnp.broadcast_to(w_col, (NA_R, WPAD))

    miota = jax.lax.broadcasted_iota(jnp.int32, (S, M), 1)
    oh_m = (mode_ref[...] == miota).astype(jnp.float32)
    dst_m, counts_m = ranks(oh_m, NA_M)
    dstm_ref[...] = dst_m

    def steps(counts_row, n_g, n_step, sg_ref, sb_ref, tot_ref):
        nt_row = jnp.floor((counts_row + float(ROWT - 1)) / float(ROWT))
        gi8 = jax.lax.broadcasted_iota(jnp.int32, (n_g, n_g), 0)
        gj8 = jax.lax.broadcasted_iota(jnp.int32, (n_g, n_g), 1)
        tri_inc = (gj8 <= gi8).astype(jnp.float32)
        iden = (gj8 == gi8).astype(jnp.float32)
        nt_col = jax.lax.dot_general(iden, nt_row, (((1,), (1,)), ((), ())),
                                     preferred_element_type=jnp.float32)
        cum_col = jax.lax.dot_general(tri_inc, nt_col,
                                      (((1,), (0,)), ((), ())),
                                      preferred_element_type=jnp.float32)
        cumprev_col = cum_col - nt_col
        total = jnp.sum(nt_col, axis=0, keepdims=True)
        s = jax.lax.broadcasted_iota(jnp.int32, (1, n_step),
                                     1).astype(jnp.float32)
        s_eff = jnp.minimum(s, jnp.maximum(total - 1.0, 0.0))
        ge = (s_eff >= cum_col).astype(jnp.float32)
        g_of_s = jnp.sum(ge, axis=0, keepdims=True)
        gi = jax.lax.broadcasted_iota(jnp.int32, (n_g, n_step),
                                      0).astype(jnp.float32)
        onehot = (gi == g_of_s).astype(jnp.float32)
        cumprev_of_s = jnp.sum(cumprev_col * onehot, axis=0, keepdims=True)
        j_of_s = s_eff - cumprev_of_s
        sg_ref[...] = g_of_s.astype(jnp.int32)
        sb_ref[...] = (g_of_s * TPG + j_of_s).astype(jnp.int32)
        tot_ref[...] = total.astype(jnp.int32)

    steps(counts_r, E, NSTEP_R, sgr_ref, sbr_ref, totr_ref)
    steps(counts_m, M, NSTEP_M, sgm_ref, sbm_ref, totm_ref)


def _run_router(flat, mode_flat, router_w, router_b):
    return pl.pallas_call(
        _router_kernel,
        grid=(1,),
        in_specs=[
            pl.BlockSpec((S, D), lambda i: (0, 0)),
            pl.BlockSpec((S, 1), lambda i: (0, 0)),
            pl.BlockSpec((E, D), lambda i: (0, 0)),
            pl.BlockSpec((E,), lambda i: (0,)),
        ],
        out_specs=[
            pl.BlockSpec((S, E), lambda i: (0, 0)),
            pl.BlockSpec((S, K), lambda i: (0, 0)),
            pl.BlockSpec((S, K), lambda i: (0, 0)),
            pl.BlockSpec((1, NSTEP_R), lambda i: (0, 0)),
            pl.BlockSpec((1, NSTEP_R), lambda i: (0, 0)),
            pl.BlockSpec((1, 1), lambda i: (0, 0)),
            pl.BlockSpec((1, NSTEP_M), lambda i: (0, 0)),
            pl.BlockSpec((1, NSTEP_M), lambda i: (0, 0)),
            pl.BlockSpec((1, 1), lambda i: (0, 0)),
            pl.BlockSpec((NA_R, 1), lambda i: (0, 0)),
            pl.BlockSpec((NA_R, WPAD), lambda i: (0, 0)),
            pl.BlockSpec((NA_M, 1), lambda i: (0, 0)),
        ],
        out_shape=[
            jax.ShapeDtypeStruct((S, E), jnp.float32),
            jax.ShapeDtypeStruct((S, K), jnp.int32),
            jax.ShapeDtypeStruct((S, K), jnp.float32),
            jax.ShapeDtypeStruct((1, NSTEP_R), jnp.int32),
            jax.ShapeDtypeStruct((1, NSTEP_R), jnp.int32),
            jax.ShapeDtypeStruct((1, 1), jnp.int32),
            jax.ShapeDtypeStruct((1, NSTEP_M), jnp.int32),
            jax.ShapeDtypeStruct((1, NSTEP_M), jnp.int32),
            jax.ShapeDtypeStruct((1, 1), jnp.int32),
            jax.ShapeDtypeStruct((NA_R, 1), jnp.int32),
            jax.ShapeDtypeStruct((NA_R, WPAD), jnp.float32),
            jax.ShapeDtypeStruct((NA_M, 1), jnp.int32),
        ],
    )(flat, mode_flat, router_w, router_b)


def _shared_ffn_kernel(x_ref, w1_ref, b1_ref, w2_ref, b2_ref, out_ref):
    h = jax.lax.dot_general(x_ref[...], w1_ref[0], (((1,), (1,)), ((), ())),
                            preferred_element_type=jnp.float32)
    h = _gelu(h + b1_ref[0])
    y = jax.lax.dot_general(h, w2_ref[0], (((1,), (1,)), ((), ())),
                            preferred_element_type=jnp.float32)
    out_ref[...] = y + b2_ref[0]


def _run_shared(flat, w1, b1, w2, b2):
    tt = 1024
    return pl.pallas_call(
        _shared_ffn_kernel,
        grid=(S // tt,),
        in_specs=[
            pl.BlockSpec((tt, D), lambda i: (i, 0)),
            pl.BlockSpec((1, H, D), lambda i: (0, 0, 0)),
            pl.BlockSpec((1, 1, H), lambda i: (0, 0, 0)),
            pl.BlockSpec((1, D, H), lambda i: (0, 0, 0)),
            pl.BlockSpec((1, 1, D), lambda i: (0, 0, 0)),
        ],
        out_specs=pl.BlockSpec((tt, D), lambda i: (i, 0)),
        out_shape=jax.ShapeDtypeStruct((S, D), jnp.float32),
    )(flat, w1, b1.reshape(1, 1, H), w2, b2.reshape(1, 1, D))


def _sc_dispatch_body(tokr_hbm, dstr_hbm, wr2d_hbm, tokm_hbm, dstm_hbm,
                      x_hbm, xgr_hbm, wrow_hbm, xgm_hbm, tb, db, wb, xb, sem):
    wid = lax.axis_index("s") * 2 + lax.axis_index("c")

    def move(tok_src, dst_src, base, xg_dst):
        pltpu.sync_copy(tok_src.at[pl.ds(base, CHUNK)], tb)
        pltpu.sync_copy(dst_src.at[pl.ds(base, CHUNK)], db)
        pltpu.async_copy(x_hbm.at[tb], xb, sem).wait()
        pltpu.async_copy(xb, xg_dst.at[db], sem).wait()

    move(tokr_hbm, dstr_hbm, wid * CHUNK, xgr_hbm)
    pltpu.sync_copy(wr2d_hbm.at[pl.ds(wid * CHUNK, CHUNK)], wb)
    pltpu.async_copy(wb, wrow_hbm.at[db], sem).wait()

    @pl.when(wid < NA_M // CHUNK)
    def _():
        move(tokm_hbm, dstm_hbm, wid * CHUNK, xgm_hbm)


def _run_sc_dispatch(tokr, dstr, wr2d, tokm, dstm, flat):
    mesh = plsc.VectorSubcoreMesh(core_axis_name="c", subcore_axis_name="s",
                                  num_cores=2, num_subcores=16)
    f = pl.kernel(
        _sc_dispatch_body,
        out_type=[
            jax.ShapeDtypeStruct((NROW_R, D), jnp.float32),
            jax.ShapeDtypeStruct((NROW_R, WPAD), jnp.float32),
            jax.ShapeDtypeStruct((NROW_M, D), jnp.float32),
        ],
        mesh=mesh,
        scratch_types=[
            pltpu.VMEM((CHUNK,), jnp.int32),
            pltpu.VMEM((CHUNK,), jnp.int32),
            pltpu.VMEM((CHUNK, WPAD), jnp.float32),
            pltpu.VMEM((CHUNK, D), jnp.float32),
            pltpu.SemaphoreType.DMA,
        ],
    )
    return f(tokr, dstr, wr2d, tokm, dstm, flat)


def _sc_combine_body(dstr_hbm, dstm_hbm, yr_hbm, ym_hbm, y0_hbm, y1_hbm,
                     ym_out_hbm, ib, yb, sem):
    wid = lax.axis_index("s") * 2 + lax.axis_index("c")
    base = wid * CH2

    def gather_back(dst_src, dst_off, y_src, out_dst):
        pltpu.sync_copy(dst_src.at[pl.ds(dst_off + base, CH2)], ib)
        pltpu.async_copy(y_src.at[ib], yb, sem).wait()
        pltpu.sync_copy(yb, out_dst.at[pl.ds(base, CH2)])

    gather_back(dstr_hbm, 0, yr_hbm, y0_hbm)
    gather_back(dstr_hbm, S, yr_hbm, y1_hbm)
    gather_back(dstm_hbm, 0, ym_hbm, ym_out_hbm)


def _run_sc_combine(dstr, dstm, y_r, y_m):
    mesh = plsc.VectorSubcoreMesh(core_axis_name="c", subcore_axis_name="s",
                                  num_cores=2, num_subcores=16)
    f = pl.kernel(
        _sc_combine_body,
        out_type=[
            jax.ShapeDtypeStruct((S, D), jnp.float32),
            jax.ShapeDtypeStruct((S, D), jnp.float32),
            jax.ShapeDtypeStruct((S, D), jnp.float32),
        ],
        mesh=mesh,
        scratch_types=[
            pltpu.VMEM((CH2,), jnp.int32),
            pltpu.VMEM((CH2, D), jnp.float32),
            pltpu.SemaphoreType.DMA,
        ],
    )
    return f(dstr, dstm, y_r, y_m)


def _grouped_ffn_kernel(sg_ref, sb_ref, tot_ref, x_ref, w1_ref, b1_ref,
                        w2_ref, b2_ref, wrow_ref, y_ref):
    s = pl.program_id(0)

    @pl.when(s < tot_ref[0])
    def _():
        h = jax.lax.dot_general(x_ref[...], w1_ref[0],
                                (((1,), (1,)), ((), ())),
                                preferred_element_type=jnp.float32)
        h = _gelu(h + b1_ref[0])
        y = jax.lax.dot_general(h, w2_ref[0], (((1,), (1,)), ((), ())),
                                preferred_element_type=jnp.float32)
        y = y + b2_ref[0]
        if wrow_ref is not None:
            y = y * wrow_ref[0, :, 0:1]
        y_ref[...] = y


def _run_grouped(xg, wrow, w1, b1, w2, b2, sg, sb, tot, n_g, n_step):
    nrow = n_g * CAP
    in_specs = [
        pl.BlockSpec((ROWT, D), lambda s, sg, sb, tot: (sb[s], 0)),
        pl.BlockSpec((1, H, D), lambda s, sg, sb, tot: (sg[s], 0, 0)),
        pl.BlockSpec((1, 1, H), lambda s, sg, sb, tot: (sg[s], 0, 0)),
        pl.BlockSpec((1, D, H), lambda s, sg, sb, tot: (sg[s], 0, 0)),
        pl.BlockSpec((1, 1, D), lambda s, sg, sb, tot: (sg[s], 0, 0)),
    ]
    args = [xg, w1, b1.reshape(n_g, 1, H), w2, b2.reshape(n_g, 1, D)]
    if wrow is not None:
        in_specs.append(
            pl.BlockSpec((1, ROWT, WPAD), lambda s, sg, sb, tot: (sb[s], 0, 0)))
        args.append(wrow.reshape(nrow // ROWT, ROWT, WPAD))
        body = _grouped_ffn_kernel
    else:
        def body(sg_ref, sb_ref, tot_ref, x_ref, w1_ref, b1_ref, w2_ref,
                 b2_ref, y_ref):
            _grouped_ffn_kernel(sg_ref, sb_ref, tot_ref, x_ref, w1_ref,
                                b1_ref, w2_ref, b2_ref, None, y_ref)
    grid_spec = pltpu.PrefetchScalarGridSpec(
        num_scalar_prefetch=3,
        grid=(n_step,),
        in_specs=in_specs,
        out_specs=pl.BlockSpec((ROWT, D), lambda s, sg, sb, tot: (sb[s], 0)),
        scratch_shapes=[],
    )
    return pl.pallas_call(
        body,
        grid_spec=grid_spec,
        out_shape=jax.ShapeDtypeStruct((nrow, D), jnp.float32),
    )(sg, sb, tot, *args)


def _combine_kernel(a_ref, b_ref, c_ref, d_ref, out_ref):
    out_ref[...] = a_ref[...] + b_ref[...] + c_ref[...] + d_ref[...]


def _run_combine(a, b, c, d):
    return pl.pallas_call(
        _combine_kernel,
        grid=(1,),
        in_specs=[pl.BlockSpec((S, D), lambda i: (0, 0))] * 4,
        out_specs=pl.BlockSpec((S, D), lambda i: (0, 0)),
        out_shape=jax.ShapeDtypeStruct((S, D), jnp.float32),
    )(a, b, c, d)


def kernel(hidden, mode_ids, router_w, router_b, shared_w1, shared_b1,
           shared_w2, shared_b2, routed_w1, routed_b1, routed_w2, routed_b2,
           mode_w1, mode_b1, mode_w2, mode_b2):
    flat = hidden.reshape(S, D)
    mode_flat = mode_ids.reshape(S, 1).astype(jnp.int32)

    (logits, tidx, tprob, sgr, sbr, totr, sgm, sbm, totm, dstr, wr2d,
     dstm) = _run_router(flat, mode_flat, router_w, router_b)

    shared_out = _run_shared(flat, shared_w1, shared_b1, shared_w2, shared_b2)

    tok1 = jnp.arange(S, dtype=jnp.int32)
    tokr = jnp.concatenate([tok1, tok1])
    dstr1 = dstr.reshape(NA_R)
    dstm1 = dstm.reshape(NA_M)

    xg_r, wrow_r, xg_m = _run_sc_dispatch(tokr, dstr1, wr2d, tok1, dstm1,
                                          flat)

    y_r = _run_grouped(xg_r, wrow_r, routed_w1, routed_b1, routed_w2,
                       routed_b2, sgr.reshape(-1), sbr.reshape(-1),
                       totr.reshape(-1), E, NSTEP_R)
    y_m = _run_grouped(xg_m, None, mode_w1, mode_b1, mode_w2, mode_b2,
                       sgm.reshape(-1), sbm.reshape(-1),
                       totm.reshape(-1), M, NSTEP_M)

    yc0, yc1, ycm = _run_sc_combine(dstr1, dstm1, y_r, y_m)

    out = _run_combine(shared_out, yc0, yc1, ycm)
    return (out.reshape(B, S, D), logits.reshape(B, S, E),
            tidx.reshape(B, S, K), tprob.reshape(B, S, K))
